# Initial kernel scaffold; baseline (speedup 1.0000x reference)
import jax
import jax.numpy as jnp
from jax.experimental import pallas as pl


def kernel(x, tables):
    raise NotImplementedError("write your pallas kernel here")


# speedup vs baseline: 3.0519x; 3.0519x over previous
import functools

import jax
import jax.numpy as jnp
import numpy as np
from jax import lax
from jax.experimental import pallas as pl
from jax.experimental.pallas import tpu as pltpu
from jax.experimental.pallas import tpu_sc as plsc

F = 26
D = 16
B = 4096
V = 3846
TOTAL = F * V
NPAIR = F * (F - 1) // 2
NC, NS = 2, 16
NW = NC * NS
BPW = B // NW

_mesh = plsc.VectorSubcoreMesh(core_axis_name="c", subcore_axis_name="s")


@functools.partial(
    pl.kernel,
    out_type=jax.ShapeDtypeStruct((B, NPAIR, D), jnp.float32),
    mesh=_mesh,
    compiler_params=pltpu.CompilerParams(use_tc_tiling_on_sc=False),
    scratch_types=[
        pltpu.VMEM((F, BPW), jnp.int32),
        pltpu.VMEM((BPW,), jnp.int32),
        pltpu.VMEM((BPW,), jnp.int32),
        pltpu.VMEM((BPW, D), jnp.float32),
        pltpu.VMEM((BPW, D), jnp.float32),
        pltpu.SemaphoreType.DMA,
        pltpu.SemaphoreType.DMA,
    ],
)
def _ffm_sc(xiT_hbm, tab_hbm, out_hbm, xi_v, idxA, idxB, bufA, bufB, semA, semB):
    wid = lax.axis_index("s") * NC + lax.axis_index("c")
    base = wid * BPW
    pltpu.sync_copy(xiT_hbm.at[:, pl.ds(base, BPW)], xi_v)

    def pair_body(i, j, p):
        jt = j * TOTAL
        it = i * TOTAL

        def idx_chunk(c, carry):
            sl = pl.ds(c * D, D)
            idxA[sl] = xi_v[i, sl] + jt
            idxB[sl] = xi_v[j, sl] + it
            return carry

        lax.fori_loop(0, BPW // D, idx_chunk, 0)
        cpA = pltpu.async_copy(tab_hbm.at[idxA], bufA, semA)
        cpB = pltpu.async_copy(tab_hbm.at[idxB], bufB, semB)
        cpA.wait()
        cpB.wait()

        def mul_row(b, carry):
            bufA[b] = bufA[b] * bufB[b]
            return carry

        lax.fori_loop(0, BPW, mul_row, 0)
        pltpu.sync_copy(bufA, out_hbm.at[pl.ds(base, BPW), p])
        return p + 1

    def i_body(i, p):
        return lax.fori_loop(i + 1, F, lambda j, pp: pair_body(i, j, pp), p)

    lax.fori_loop(0, F - 1, i_body, 0)


def kernel(x, tables):
    offs = jnp.asarray(np.arange(F, dtype=np.int32) * V)
    xiT = jnp.transpose(x.astype(jnp.int32) + offs[None, :])
    tab = tables.reshape(F * TOTAL, D)
    return _ffm_sc(xiT, tab)


# speedup vs baseline: 3.1471x; 1.0312x over previous
import functools

import jax
import jax.numpy as jnp
import numpy as np
from jax import lax
from jax.experimental import pallas as pl
from jax.experimental.pallas import tpu as pltpu
from jax.experimental.pallas import tpu_sc as plsc

F = 26
D = 16
B = 4096
V = 3846
TOTAL = F * V
NPAIR = F * (F - 1) // 2
NC, NS = 2, 16
NW = NC * NS
BPW = B // NW
NBUF = 4
LAG = 3

_mesh = plsc.VectorSubcoreMesh(core_axis_name="c", subcore_axis_name="s")


@functools.partial(
    pl.kernel,
    out_type=jax.ShapeDtypeStruct((B, NPAIR, D), jnp.float32),
    mesh=_mesh,
    compiler_params=pltpu.CompilerParams(use_tc_tiling_on_sc=False),
    scratch_types=[
        pltpu.VMEM((F, BPW), jnp.int32),
        pltpu.VMEM((NBUF, BPW), jnp.int32),
        pltpu.VMEM((NBUF, BPW), jnp.int32),
        pltpu.VMEM((NBUF, BPW, D), jnp.float32),
        pltpu.VMEM((NBUF, BPW, D), jnp.float32),
        pltpu.SemaphoreType.DMA((NBUF,)),
        pltpu.SemaphoreType.DMA((NBUF,)),
    ],
)
def _ffm_sc(xiT_hbm, tab_hbm, out_hbm, xi_v, idxA, idxB, bufA, bufB, sg, so):
    wid = lax.axis_index("s") * NC + lax.axis_index("c")
    base = wid * BPW
    pltpu.sync_copy(xiT_hbm.at[:, pl.ds(base, BPW)], xi_v)

    def body(p, carry):
        i, j = carry
        k = lax.rem(p, NBUF)

        @pl.when(p >= NBUF)
        def _():
            pltpu.make_async_copy(
                bufA.at[k], out_hbm.at[pl.ds(base, BPW), p - NBUF], so.at[k]
            ).wait()

        @pl.when(p < NPAIR)
        def _():
            jt = j * TOTAL
            it = i * TOTAL
            for c in range(BPW // D):
                sl = pl.ds(c * D, D)
                idxA[k, sl] = xi_v[i, sl] + jt
                idxB[k, sl] = xi_v[j, sl] + it
            pltpu.async_copy(tab_hbm.at[idxA.at[k]], bufA.at[k], sg.at[k])
            pltpu.async_copy(tab_hbm.at[idxB.at[k]], bufB.at[k], sg.at[k])

        @pl.when(p >= LAG)
        def _():
            q = p - LAG
            k2 = lax.rem(q, NBUF)
            pltpu.make_async_copy(tab_hbm.at[idxA.at[k2]], bufA.at[k2], sg.at[k2]).wait()
            pltpu.make_async_copy(tab_hbm.at[idxB.at[k2]], bufB.at[k2], sg.at[k2]).wait()

            def mul_rows(c, carry2):
                for u in range(8):
                    b = c * 8 + u
                    bufA[k2, b] = bufA[k2, b] * bufB[k2, b]
                return carry2

            lax.fori_loop(0, BPW // 8, mul_rows, 0)
            pltpu.async_copy(bufA.at[k2], out_hbm.at[pl.ds(base, BPW), q], so.at[k2])

        adv = j == (F - 1)
        ni = jnp.where(adv, i + 1, i)
        nj = jnp.where(adv, ni + 1, j + 1)
        return ni, nj

    lax.fori_loop(0, NPAIR + LAG, body, (jnp.int32(0), jnp.int32(1)))

    qlast = NPAIR - 1
    klast = qlast % NBUF
    pltpu.make_async_copy(
        bufA.at[klast], out_hbm.at[pl.ds(base, BPW), qlast], so.at[klast]
    ).wait()


def kernel(x, tables):
    offs = jnp.asarray(np.arange(F, dtype=np.int32) * V)
    xiT = jnp.transpose(x.astype(jnp.int32) + offs[None, :])
    tab = tables.reshape(F * TOTAL, D)
    return _ffm_sc(xiT, tab)


# speedup vs baseline: 59.0146x; 18.7518x over previous
import functools

import jax
import jax.numpy as jnp
import numpy as np
from jax import lax
from jax.experimental import pallas as pl
from jax.experimental.pallas import tpu as pltpu
from jax.experimental.pallas import tpu_sc as plsc

F = 26
D = 16
B = 4096
V = 3846
NPAIR = F * (F - 1) // 2
NC, NS = 2, 16
NW = NC * NS
DH = D // 2
W = 4096
VPAD = 100096
XC = 256
CB = 128
BASE = NPAIR // NW
EXTRA = NPAIR - BASE * NW

_mesh = plsc.VectorSubcoreMesh(core_axis_name="c", subcore_axis_name="s")


@functools.partial(
    pl.kernel,
    out_type=jax.ShapeDtypeStruct((NPAIR, D, B), jnp.float32),
    mesh=_mesh,
    compiler_params=pltpu.CompilerParams(
        use_tc_tiling_on_sc=True, needs_layout_passes=False
    ),
    scratch_types=[
        pltpu.VMEM((DH, W), jnp.float32),
        pltpu.VMEM((DH, W), jnp.float32),
        pltpu.VMEM((2, DH, CB), jnp.float32),
        pltpu.VMEM((2, XC), jnp.int32),
        pltpu.VMEM((2, XC), jnp.int32),
        pltpu.SemaphoreType.DMA,
        pltpu.SemaphoreType.DMA((2,)),
        pltpu.SemaphoreType.DMA((2,)),
    ],
)
def _ffm_sc(xf_hbm, tabT_hbm, out_hbm, BA, BB, OS, XA, XB, sblk, so, sx):
    wid = lax.axis_index("s") * NC + lax.axis_index("c")
    p0 = wid * BASE + jnp.minimum(wid, EXTRA)
    npairs = BASE + jnp.where(wid < EXTRA, 1, 0)
    p1 = p0 + npairs

    def tri_body(i, st):
        i0, s = st
        nb = s + (F - 1 - i)
        c = jnp.logical_and(i == i0, nb <= p0)
        return jnp.where(c, i + 1, i0), jnp.where(c, nb, s)

    i0, s0 = lax.fori_loop(0, F - 1, tri_body, (jnp.int32(0), jnp.int32(0)))
    j0 = i0 + 1 + (p0 - s0)

    def pair_body(p, carry):
        i, j = carry
        offa = i * V
        offb = j * V
        cola = jnp.minimum((offa // 128) * 128, VPAD - W)
        colb = jnp.minimum((offb // 128) * 128, VPAD - W)
        ra = offa - cola
        rb = offb - colb

        for h in range(D // DH):
            cpa = pltpu.async_copy(
                tabT_hbm.at[j, pl.ds(h * DH, DH), pl.ds(cola, W)], BA, sblk
            )
            cpb = pltpu.async_copy(
                tabT_hbm.at[i, pl.ds(h * DH, DH), pl.ds(colb, W)], BB, sblk
            )
            pltpu.async_copy(xf_hbm.at[pl.ds(i * B, XC)], XA.at[0], sx.at[0])
            pltpu.async_copy(xf_hbm.at[pl.ds(j * B, XC)], XB.at[0], sx.at[0])
            cpa.wait()
            cpb.wait()

            def sc_body(sc, _):
                slot = lax.rem(sc, 2)
                nxt = lax.rem(sc + 1, 2)

                @pl.when(sc + 1 < B // XC)
                def _():
                    pltpu.async_copy(
                        xf_hbm.at[pl.ds(i * B + (sc + 1) * XC, XC)],
                        XA.at[nxt], sx.at[nxt],
                    )
                    pltpu.async_copy(
                        xf_hbm.at[pl.ds(j * B + (sc + 1) * XC, XC)],
                        XB.at[nxt], sx.at[nxt],
                    )

                pltpu.make_async_copy(
                    xf_hbm.at[pl.ds(0, XC)], XA.at[slot], sx.at[slot]
                ).wait()
                pltpu.make_async_copy(
                    xf_hbm.at[pl.ds(0, XC)], XB.at[slot], sx.at[slot]
                ).wait()

                def cc_body(cc, _):
                    os = lax.rem(cc, 2)
                    g = (((p - p0) * 2 + h) * (B // CB)) + sc * (XC // CB) + cc

                    @pl.when(g >= 2)
                    def _():
                        pltpu.make_async_copy(
                            OS.at[os], out_hbm.at[0, pl.ds(0, DH), pl.ds(0, CB)],
                            so.at[os],
                        ).wait()

                    for s in range(CB // D):
                        va = XA[slot, pl.ds(cc * CB + s * D, D)] + ra
                        vb = XB[slot, pl.ds(cc * CB + s * D, D)] + rb
                        for d in range(DH):
                            dv = jnp.full((D,), d, jnp.int32)
                            a = plsc.load_gather(BA, [dv, va])
                            b = plsc.load_gather(BB, [dv, vb])
                            OS[os, d, pl.ds(s * D, D)] = a * b

                    pltpu.async_copy(
                        OS.at[os],
                        out_hbm.at[p, pl.ds(h * DH, DH), pl.ds(sc * XC + cc * CB, CB)],
                        so.at[os],
                    )
                    return 0

                lax.fori_loop(0, XC // CB, cc_body, 0)
                return 0

            lax.fori_loop(0, B // XC, sc_body, 0)

        adv = j == (F - 1)
        ni = jnp.where(adv, i + 1, i)
        nj = jnp.where(adv, ni + 1, j + 1)
        return ni, nj

    lax.fori_loop(p0, p1, pair_body, (i0, j0))

    for os in range(2):
        pltpu.make_async_copy(
            OS.at[os], out_hbm.at[0, pl.ds(0, DH), pl.ds(0, CB)], so.at[os]
        ).wait()


def kernel(x, tables):
    tabT = jnp.transpose(tables, (0, 2, 1))
    xf = jnp.transpose(x.astype(jnp.int32)).reshape(F * B)
    out3 = _ffm_sc(xf, tabT)
    return jnp.transpose(out3, (2, 0, 1))
